# baseline (speedup 1.0000x reference)
import jax
import jax.numpy as jnp
from jax.experimental import pallas as pl
from jax.experimental.pallas import tpu as pltpu

NQ = 8
K = 1024
D = 256
TN = 1024


def _rvq_body(x_ref, cb_ref, zq_ref, codes_ref, loss_ref, c2_ref):
    i = pl.program_id(0)

    @pl.when(i == 0)
    def _init():
        loss_ref[...] = jnp.zeros_like(loss_ref)
        c2_ref[...] = jnp.sum(cb_ref[...] * cb_ref[...], axis=2)

    r = x_ref[...]
    zq = jnp.zeros_like(r)
    iota = jax.lax.broadcasted_iota(jnp.int32, (TN, K), 1)
    idxs = []
    losses = []
    for q in range(NQ):
        cb = cb_ref[q]
        mm = jax.lax.dot_general(
            r, cb, (((1,), (1,)), ((), ())),
            preferred_element_type=jnp.float32)
        r2 = jnp.sum(r * r, axis=1, keepdims=True)
        d2 = (r2 - 2.0 * mm) + c2_ref[q][None, :]
        m = jnp.min(d2, axis=1, keepdims=True)
        idx = jnp.min(jnp.where(d2 == m, iota, K), axis=1)
        onehot = (iota == idx[:, None]).astype(jnp.float32)
        quant = jax.lax.dot_general(
            onehot, cb, (((1,), (0,)), ((), ())),
            preferred_element_type=jnp.float32,
            precision=jax.lax.Precision.HIGHEST)
        zq = zq + quant
        r = r - quant
        losses.append(jnp.sum(r * r))
        idxs.append(idx)
    zq_ref[...] = zq
    codes_ref[...] = jnp.stack(idxs, axis=0)
    loss_ref[...] += jnp.concatenate(
        [l.reshape(1, 1) for l in losses], axis=1)


def kernel(latent, codebooks):
    Bm, Tm, Dm = latent.shape
    N = Bm * Tm
    x = latent.reshape(N, Dm)
    grid = (N // TN,)
    zq, codes_t, loss = pl.pallas_call(
        _rvq_body,
        grid=grid,
        in_specs=[
            pl.BlockSpec((TN, D), lambda i: (i, 0)),
            pl.BlockSpec((NQ, K, D), lambda i: (0, 0, 0)),
        ],
        out_specs=[
            pl.BlockSpec((TN, D), lambda i: (i, 0)),
            pl.BlockSpec((NQ, TN), lambda i: (0, i)),
            pl.BlockSpec((1, NQ), lambda i: (0, 0)),
        ],
        out_shape=[
            jax.ShapeDtypeStruct((N, D), jnp.float32),
            jax.ShapeDtypeStruct((NQ, N), jnp.int32),
            jax.ShapeDtypeStruct((1, NQ), jnp.float32),
        ],
        scratch_shapes=[pltpu.VMEM((NQ, K), jnp.float32)],
        compiler_params=pltpu.CompilerParams(
            dimension_semantics=("arbitrary",)),
    )(x, codebooks)
    z_q = zq.reshape(Bm, Tm, Dm)
    codes = codes_t.T.reshape(Bm, Tm, NQ)
    q_loss = jnp.sum(loss) / (N * Dm)
    return z_q, codes, q_loss


# speedup vs baseline: 1.9036x; 1.9036x over previous
import jax
import jax.numpy as jnp
from jax.experimental import pallas as pl
from jax.experimental.pallas import tpu as pltpu

NQ = 8
K = 1024
D = 256
TN = 1024


def _rvq_body(x_ref, cb_ref, cbh_ref, cbm_ref, cbl_ref,
              zq_ref, codes_ref, loss_ref, c2_ref):
    i = pl.program_id(0)

    @pl.when(i == 0)
    def _init():
        loss_ref[...] = jnp.zeros_like(loss_ref)
        c2_ref[...] = jnp.sum(cb_ref[...] * cb_ref[...], axis=2)

    r = x_ref[...]
    zq = jnp.zeros_like(r)
    iota = jax.lax.broadcasted_iota(jnp.int32, (TN, K), 1)
    idxs = []
    losses = []
    for q in range(NQ):
        cb = cb_ref[q]
        mm = jax.lax.dot_general(
            r, cb, (((1,), (1,)), ((), ())),
            preferred_element_type=jnp.float32)
        r2 = jnp.sum(r * r, axis=1, keepdims=True)
        d2 = (r2 - 2.0 * mm) + c2_ref[q][None, :]
        m = jnp.min(d2, axis=1, keepdims=True)
        idx = jnp.min(jnp.where(d2 == m, iota, K), axis=1)
        onehot = (iota == idx[:, None]).astype(jnp.bfloat16)
        gdims = (((1,), (0,)), ((), ()))
        quant = (jax.lax.dot_general(onehot, cbh_ref[q], gdims,
                                     preferred_element_type=jnp.float32)
                 + jax.lax.dot_general(onehot, cbm_ref[q], gdims,
                                       preferred_element_type=jnp.float32)
                 + jax.lax.dot_general(onehot, cbl_ref[q], gdims,
                                       preferred_element_type=jnp.float32))
        zq = zq + quant
        r = r - quant
        losses.append(jnp.sum(r * r))
        idxs.append(idx)
    zq_ref[...] = zq
    codes_ref[...] = jnp.stack(idxs, axis=0)
    loss_ref[...] += jnp.concatenate(
        [l.reshape(1, 1) for l in losses], axis=1)


def kernel(latent, codebooks):
    Bm, Tm, Dm = latent.shape
    N = Bm * Tm
    x = latent.reshape(N, Dm)
    cb_hi = codebooks.astype(jnp.bfloat16)
    r1 = codebooks - cb_hi.astype(jnp.float32)
    cb_mid = r1.astype(jnp.bfloat16)
    cb_lo = (r1 - cb_mid.astype(jnp.float32)).astype(jnp.bfloat16)
    grid = (N // TN,)
    zq, codes_t, loss = pl.pallas_call(
        _rvq_body,
        grid=grid,
        in_specs=[
            pl.BlockSpec((TN, D), lambda i: (i, 0)),
            pl.BlockSpec((NQ, K, D), lambda i: (0, 0, 0)),
            pl.BlockSpec((NQ, K, D), lambda i: (0, 0, 0)),
            pl.BlockSpec((NQ, K, D), lambda i: (0, 0, 0)),
            pl.BlockSpec((NQ, K, D), lambda i: (0, 0, 0)),
        ],
        out_specs=[
            pl.BlockSpec((TN, D), lambda i: (i, 0)),
            pl.BlockSpec((NQ, TN), lambda i: (0, i)),
            pl.BlockSpec((1, NQ), lambda i: (0, 0)),
        ],
        out_shape=[
            jax.ShapeDtypeStruct((N, D), jnp.float32),
            jax.ShapeDtypeStruct((NQ, N), jnp.int32),
            jax.ShapeDtypeStruct((1, NQ), jnp.float32),
        ],
        scratch_shapes=[pltpu.VMEM((NQ, K), jnp.float32)],
        compiler_params=pltpu.CompilerParams(
            dimension_semantics=("arbitrary",)),
    )(x, codebooks, cb_hi, cb_mid, cb_lo)
    z_q = zq.reshape(Bm, Tm, Dm)
    codes = codes_t.T.reshape(Bm, Tm, NQ)
    q_loss = jnp.sum(loss) / (N * Dm)
    return z_q, codes, q_loss
